# Initial kernel scaffold; baseline (speedup 1.0000x reference)
import jax
import jax.numpy as jnp
from jax.experimental import pallas as pl


def kernel(features_omics1, features_omics2, adj_spatial_omics1, adj_feature_omics1, adj_spatial_omics2, adj_feature_omics2, params):
    raise NotImplementedError("write your pallas kernel here")


# speedup vs baseline: 1.1671x; 1.1671x over previous
import functools

import jax
import jax.numpy as jnp
from jax.experimental import pallas as pl
from jax.experimental.pallas import tpu as pltpu

N = 4096
F32 = jnp.float32
BF = jnp.bfloat16
_VMEM_LIMIT = 100 * 1024 * 1024


def _dot(a, b):
    return jax.lax.dot(a, b, preferred_element_type=F32)


_dot_fast = _dot


def _bn(t, m, v, g, b):
    return g * (t - m) / jnp.sqrt(v + 1e-5) + b


def _mm_kernel(x_ref, w_ref, y_ref):
    y_ref[...] = _dot(x_ref[...], w_ref[...])


def _mm(x, w, T=512):
    din, h = w.shape
    return pl.pallas_call(
        _mm_kernel,
        grid=(N // T,),
        in_specs=[pl.BlockSpec((T, din), lambda i: (i, 0)),
                  pl.BlockSpec((din, h), lambda i: (0, 0))],
        out_specs=pl.BlockSpec((T, h), lambda i: (i, 0)),
        out_shape=jax.ShapeDtypeStruct((N, h), F32),
    )(x, w)


def _enc_kernel(T, H, z1_ref, adj_ref, g_ref, b_ref, w2_ref,
                z_ref, t, U, ssum):
    p = pl.program_id(0)
    i = pl.program_id(1)
    row = pl.ds(i * T, T)

    @pl.when(p == 0)
    def _():
        @pl.when(i == 0)
        def _():
            ssum[...] = jnp.zeros_like(ssum)
        ti = jnp.maximum(_dot(adj_ref[...], z1_ref[...]), 0.0)
        t[row, :] = ti
        ssum[...] += jnp.sum(ti, axis=0, keepdims=True)

    @pl.when(p == 1)
    def _():
        @pl.when(i == 0)
        def _():
            m = jnp.mean(t[...], axis=0, keepdims=True)
            v = jnp.mean((t[...] - m) ** 2, axis=0, keepdims=True)
            xn = _bn(t[...], m, v, g_ref[...], b_ref[...])
            U[...] = _dot(xn, w2_ref[...])
        z_ref[...] = _dot(adj_ref[...], U[...])


def _encoder(z1, adj, g, b, w2, T=256):
    h = w2.shape[0]
    r = N // T
    return pl.pallas_call(
        functools.partial(_enc_kernel, T, h),
        grid=(2, r),
        in_specs=[
            pl.BlockSpec((N, h), lambda p, i: (0, 0)),
            pl.BlockSpec((T, N), lambda p, i: (i, 0)),
            pl.BlockSpec((1, h), lambda p, i: (0, 0)),
            pl.BlockSpec((1, h), lambda p, i: (0, 0)),
            pl.BlockSpec((h, 128), lambda p, i: (0, 0)),
        ],
        out_specs=pl.BlockSpec((T, 128),
                               lambda p, i: (jnp.where(p == 1, i, 0), 0)),
        out_shape=jax.ShapeDtypeStruct((N, 128), F32),
        scratch_shapes=[
            pltpu.VMEM((N, h), F32),
            pltpu.VMEM((N, 128), F32),
            pltpu.VMEM((1, h), F32),
        ],
        compiler_params=pltpu.CompilerParams(
            dimension_semantics=("arbitrary", "arbitrary"),
            vmem_limit_bytes=_VMEM_LIMIT,
        ),
    )(z1, adj, g.reshape(1, h), b.reshape(1, h), w2)


def _rt(x):
    return x.astype(BF).astype(F32)


def _attn_pair(e1, e2, w, u):
    s1 = _dot(jnp.tanh(_dot(e1, w)), u)
    s2 = _dot(jnp.tanh(_dot(e2, w)), u)
    u1 = s1 + 1e-6
    u2 = s2 + 1e-6
    mx = jnp.maximum(u1, u2)
    e1s = jnp.exp(u1 - mx)
    e2s = jnp.exp(u2 - mx)
    den = e1s + e2s
    a1 = e1s / den
    a2 = e2s / den
    comb = _rt(e1) * _rt(a1) + _rt(e2) * _rt(a2)
    return comb, a1, a2


def _attn_kernel(zs1_ref, zf1_ref, zs2_ref, zf2_ref, w1_ref, u1_ref,
                 w2_ref, u2_ref, wc_ref, uc_ref,
                 z1_ref, z2_ref, zc_ref, a1_ref, a2_ref, ac_ref):
    z1, p1, q1 = _attn_pair(zs1_ref[...], zf1_ref[...], w1_ref[...], u1_ref[...])
    z2, p2, q2 = _attn_pair(zs2_ref[...], zf2_ref[...], w2_ref[...], u2_ref[...])
    zc, pc, qc = _attn_pair(z1, z2, wc_ref[...], uc_ref[...])
    z1_ref[...] = z1
    z2_ref[...] = z2
    zc_ref[...] = zc
    a1_ref[...] = jnp.concatenate([p1, q1], axis=1)
    a2_ref[...] = jnp.concatenate([p2, q2], axis=1)
    ac_ref[...] = jnp.concatenate([pc, qc], axis=1)


def _attention(zs1, zf1, zs2, zf2, params, T=512):
    r = N // T
    d = 128
    zspec = pl.BlockSpec((T, d), lambda i: (i, 0))
    wspec = pl.BlockSpec((d, d), lambda i: (0, 0))
    uspec = pl.BlockSpec((d, 1), lambda i: (0, 0))
    aspec = pl.BlockSpec((T, 2), lambda i: (i, 0))
    return pl.pallas_call(
        _attn_kernel,
        grid=(r,),
        in_specs=[zspec, zspec, zspec, zspec,
                  wspec, uspec, wspec, uspec, wspec, uspec],
        out_specs=[zspec, zspec, zspec, aspec, aspec, aspec],
        out_shape=[
            jax.ShapeDtypeStruct((N, d), F32),
            jax.ShapeDtypeStruct((N, d), F32),
            jax.ShapeDtypeStruct((N, d), F32),
            jax.ShapeDtypeStruct((N, 2), F32),
            jax.ShapeDtypeStruct((N, 2), F32),
            jax.ShapeDtypeStruct((N, 2), F32),
        ],
        compiler_params=pltpu.CompilerParams(
            dimension_semantics=("arbitrary",),
        ),
    )(zs1, zf1, zs2, zf2, params['a1_w'], params['a1_u'],
      params['a2_w'], params['a2_u'], params['ac_w'], params['ac_u'])


def _dec_kernel(T, Hd, Dd, He,
                v1_ref, v2_ref, adj_ref, gd_ref, bd_ref, wd2_ref,
                we1_ref, ge_ref, be_ref, we2_ref,
                r_ref, x_ref,
                SV, St1, St2, SP1, SP2,
                ssum1, ssum2, ssum3):
    p = pl.program_id(0)
    i = pl.program_id(1)
    row = pl.ds(i * T, T)

    @pl.when(p == 0)
    def _():
        @pl.when(i == 0)
        def _():
            for ref in (ssum1, ssum2, ssum3):
                ref[...] = jnp.zeros_like(ref)
        a = adj_ref[...]
        t1 = jnp.maximum(_dot(a, v1_ref[...]), 0.0)
        St1[row, :] = t1
        ssum1[...] += jnp.sum(t1, axis=0, keepdims=True)
        t2 = jnp.maximum(_dot(a, v2_ref[...]), 0.0)
        St2[row, :] = t2
        ssum2[...] += jnp.sum(t2, axis=0, keepdims=True)

    @pl.when(p == 1)
    def _():
        @pl.when(i == 0)
        def _():
            m1 = jnp.mean(St1[...], axis=0, keepdims=True)
            v1 = jnp.mean((St1[...] - m1) ** 2, axis=0, keepdims=True)
            xn1 = _bn(St1[...], m1, v1, gd_ref[...], bd_ref[...])
            SP1[...] = _dot(xn1, wd2_ref[...])
            m2 = jnp.mean(St2[...], axis=0, keepdims=True)
            v2 = jnp.mean((St2[...] - m2) ** 2, axis=0, keepdims=True)
            xn2 = _bn(St2[...], m2, v2, gd_ref[...], bd_ref[...])
            SP2[...] = _dot(xn2, wd2_ref[...])
        s_i = _dot(adj_ref[...], SP2[...])
        SV[row, :] = _dot(s_i, we1_ref[...])

    @pl.when(p == 2)
    def _():
        t3 = jnp.maximum(_dot(adj_ref[...], SV[...]), 0.0)
        St1[row, :] = t3
        ssum3[...] += jnp.sum(t3, axis=0, keepdims=True)

    @pl.when(p == 3)
    def _():
        @pl.when(i == 0)
        def _():
            m3 = jnp.mean(St1[...], axis=0, keepdims=True)
            v3 = jnp.mean((St1[...] - m3) ** 2, axis=0, keepdims=True)
            xn3 = _bn(St1[...], m3, v3, ge_ref[...], be_ref[...])
            SP2[:, 0:128] = _dot(xn3, we2_ref[...])
        a = adj_ref[...]
        r_ref[...] = _dot_fast(a, SP1[...])
        x_ref[...] = _dot_fast(a, SP2[:, 0:128])


def _decoder_chain(v1, v2, adj, gd, bd, wd2, we1, ge, be, we2, T=128):
    hd = wd2.shape[0]
    dd = wd2.shape[1]
    he = we1.shape[1]
    r = N // T
    return pl.pallas_call(
        functools.partial(_dec_kernel, T, hd, dd, he),
        grid=(4, r),
        in_specs=[
            pl.BlockSpec((N, hd), lambda p, i: (0, 0)),
            pl.BlockSpec((N, hd), lambda p, i: (0, 0)),
            pl.BlockSpec((T, N), lambda p, i: (i, 0)),
            pl.BlockSpec((1, hd), lambda p, i: (0, 0)),
            pl.BlockSpec((1, hd), lambda p, i: (0, 0)),
            pl.BlockSpec((hd, dd), lambda p, i: (0, 0)),
            pl.BlockSpec((dd, he), lambda p, i: (0, 0)),
            pl.BlockSpec((1, he), lambda p, i: (0, 0)),
            pl.BlockSpec((1, he), lambda p, i: (0, 0)),
            pl.BlockSpec((he, 128), lambda p, i: (0, 0)),
        ],
        out_specs=[
            pl.BlockSpec((T, dd), lambda p, i: (jnp.where(p == 3, i, 0), 0)),
            pl.BlockSpec((T, 128), lambda p, i: (jnp.where(p == 3, i, 0), 0)),
        ],
        out_shape=[
            jax.ShapeDtypeStruct((N, dd), F32),
            jax.ShapeDtypeStruct((N, 128), F32),
        ],
        scratch_shapes=[
            pltpu.VMEM((N, he), F32),
            pltpu.VMEM((N, hd), F32),
            pltpu.VMEM((N, hd), F32),
            pltpu.VMEM((N, dd), F32),
            pltpu.VMEM((N, dd), F32),
            pltpu.VMEM((1, hd), F32),
            pltpu.VMEM((1, hd), F32),
            pltpu.VMEM((1, he), F32),
        ],
        compiler_params=pltpu.CompilerParams(
            dimension_semantics=("arbitrary", "arbitrary"),
            vmem_limit_bytes=_VMEM_LIMIT,
        ),
    )(v1, v2, adj, gd.reshape(1, hd), bd.reshape(1, hd), wd2,
      we1, ge.reshape(1, he), be.reshape(1, he), we2)


def kernel(features_omics1, features_omics2, adj_spatial_omics1,
           adj_feature_omics1, adj_spatial_omics2, adj_feature_omics2,
           params):
    p = params
    z1e1 = _mm(features_omics1, p['e1_W1'])
    z1e2 = _mm(features_omics2, p['e2_W1'])
    z_sp1 = _encoder(z1e1, adj_spatial_omics1, p['e1_g'], p['e1_b'], p['e1_W2'])
    z_ft1 = _encoder(z1e1, adj_feature_omics1, p['e1_g'], p['e1_b'], p['e1_W2'])
    z_sp2 = _encoder(z1e2, adj_spatial_omics2, p['e2_g'], p['e2_b'], p['e2_W2'])
    z_ft2 = _encoder(z1e2, adj_feature_omics2, p['e2_g'], p['e2_b'], p['e2_W2'])

    z1, z2, zc, a1, a2, ac = _attention(z_sp1, z_ft1, z_sp2, z_ft2, p)

    r1, x21 = _decoder_chain(_mm(zc, p['d1_W1']), _mm(z2, p['d1_W1']),
                             adj_spatial_omics1,
                             p['d1_g'], p['d1_b'], p['d1_W2'],
                             p['e1_W1'], p['e1_g'], p['e1_b'], p['e1_W2'])
    r2, x12 = _decoder_chain(_mm(zc, p['d2_W1']), _mm(z1, p['d2_W1']),
                             adj_spatial_omics2,
                             p['d2_g'], p['d2_b'], p['d2_W2'],
                             p['e2_W1'], p['e2_g'], p['e2_b'], p['e2_W2'])

    return (z1, z2, zc, r1, r2, x12, x21, a1, a2, ac)
